# baseline (speedup 1.0000x reference)
import functools

import jax
import jax.numpy as jnp
from jax import lax
from jax.experimental import pallas as pl
from jax.experimental.pallas import tpu as pltpu
from jax.experimental.pallas import tpu_sc as plsc

SEQ_LEN = 8192
D_MODEL = 768
BATCH = 4

NB = BATCH * SEQ_LEN
NC = 2
NS = 16
NW = NC * NS
BPW = NB // NW
CH = 64
NCHUNK = BPW // CH

_mesh = plsc.VectorSubcoreMesh(core_axis_name="c", subcore_axis_name="s")


@functools.partial(
    pl.kernel,
    out_type=jax.ShapeDtypeStruct((NB, D_MODEL), jnp.float32),
    mesh=_mesh,
    scratch_types=[
        pltpu.VMEM((BPW,), jnp.int32),
        pltpu.VMEM((CH, D_MODEL), jnp.float32),
        pltpu.VMEM((CH, D_MODEL), jnp.float32),
        pltpu.SemaphoreType.DMA,
        pltpu.SemaphoreType.DMA,
        pltpu.SemaphoreType.DMA,
        pltpu.SemaphoreType.DMA,
    ],
)
def _emb_lookup(idx_hbm, table_hbm, out_hbm, idx_v, buf0, buf1,
                gsem0, gsem1, wsem0, wsem1):
    wid = lax.axis_index("s") * NC + lax.axis_index("c")
    base = wid * BPW
    pltpu.sync_copy(idx_hbm.at[pl.ds(base, BPW)], idx_v)

    bufs = (buf0, buf1)
    gsems = (gsem0, gsem1)
    wsems = (wsem0, wsem1)

    def gather(g):
        j = g & 1
        return pltpu.async_copy(
            table_hbm.at[idx_v.at[pl.ds(g * CH, CH)]], bufs[j], gsems[j])

    def writeback(g):
        j = g & 1
        return pltpu.async_copy(
            bufs[j], out_hbm.at[pl.ds(base + g * CH, CH)], wsems[j])

    hg = {0: gather(0), 1: gather(1)}
    hw = {}
    for g in range(NCHUNK):
        hg[g].wait()
        hw[g] = writeback(g)
        if 1 <= g <= NCHUNK - 2:
            hw[g - 1].wait()
            hg[g + 1] = gather(g + 1)
    hw[NCHUNK - 2].wait()
    hw[NCHUNK - 1].wait()


def kernel(x, table):
    idx = x.reshape(NB).astype(jnp.int32)
    out = _emb_lookup(idx, table)
    return out.reshape(BATCH, SEQ_LEN, D_MODEL)


# speedup vs baseline: 1.0356x; 1.0356x over previous
import functools

import jax
import jax.numpy as jnp
from jax import lax
from jax.experimental import pallas as pl
from jax.experimental.pallas import tpu as pltpu
from jax.experimental.pallas import tpu_sc as plsc

SEQ_LEN = 8192
D_MODEL = 768
BATCH = 4

NB = BATCH * SEQ_LEN
NC = 2
NS = 16
NW = NC * NS
BPW = NB // NW
CH = 64
NCHUNK = BPW // CH

_mesh = plsc.VectorSubcoreMesh(core_axis_name="c", subcore_axis_name="s")


@functools.partial(
    pl.kernel,
    out_type=jax.ShapeDtypeStruct((NB, D_MODEL), jnp.float32),
    mesh=_mesh,
    scratch_types=[
        pltpu.VMEM((BPW,), jnp.int32),
        pltpu.VMEM((CH, D_MODEL), jnp.float32),
        pltpu.VMEM((CH, D_MODEL), jnp.float32),
        pltpu.SemaphoreType.DMA,
        pltpu.SemaphoreType.DMA,
    ],
)
def _emb_lookup(idx_hbm, table_hbm, out_hbm, idx_v, buf0, buf1, sem0, sem1):
    wid = lax.axis_index("s") * NC + lax.axis_index("c")
    base = wid * BPW
    pltpu.sync_copy(idx_hbm.at[pl.ds(base, BPW)], idx_v)

    bufs = (buf0, buf1)
    sems = (sem0, sem1)

    def gather(g, j):
        off = pl.multiple_of(g * CH, CH)
        pltpu.async_copy(table_hbm.at[idx_v.at[pl.ds(off, CH)]], bufs[j], sems[j])

    def drain_and_store(g, j):
        pltpu.make_async_copy(table_hbm.at[pl.ds(0, CH)], bufs[j], sems[j]).wait()
        off = pl.multiple_of(g * CH, CH)
        pltpu.sync_copy(bufs[j], out_hbm.at[pl.ds(base + off, CH)])

    gather(0, 0)

    @pl.loop(0, NCHUNK, step=2)
    def _pair(g):
        gather(g + 1, 1)
        drain_and_store(g, 0)

        @pl.when(g + 2 < NCHUNK)
        def _():
            gather(g + 2, 0)

        drain_and_store(g + 1, 1)


def kernel(x, table):
    idx = x.reshape(NB).astype(jnp.int32)
    out = _emb_lookup(idx, table)
    return out.reshape(BATCH, SEQ_LEN, D_MODEL)


# speedup vs baseline: 1.0375x; 1.0018x over previous
import functools

import jax
import jax.numpy as jnp
from jax import lax
from jax.experimental import pallas as pl
from jax.experimental.pallas import tpu as pltpu
from jax.experimental.pallas import tpu_sc as plsc

SEQ_LEN = 8192
D_MODEL = 768
BATCH = 4

NB = BATCH * SEQ_LEN
NC = 2
NS = 16
NW = NC * NS
BPW = NB // NW
CH = 64
NCHUNK = BPW // CH

_mesh = plsc.VectorSubcoreMesh(core_axis_name="c", subcore_axis_name="s")


@functools.partial(
    pl.kernel,
    out_type=jax.ShapeDtypeStruct((BATCH, SEQ_LEN, D_MODEL), jnp.float32),
    mesh=_mesh,
    scratch_types=[
        pltpu.VMEM((BPW,), jnp.int32),
        pltpu.VMEM((CH, D_MODEL), jnp.float32),
        pltpu.VMEM((CH, D_MODEL), jnp.float32),
        pltpu.SemaphoreType.DMA,
        pltpu.SemaphoreType.DMA,
    ],
)
def _emb_lookup(idx_hbm, table_hbm, out_hbm, idx_v, buf0, buf1, sem0, sem1):
    wid = lax.axis_index("s") * NC + lax.axis_index("c")
    b = wid * BPW // SEQ_LEN
    s0 = wid * BPW % SEQ_LEN
    pltpu.sync_copy(idx_hbm.at[b, pl.ds(s0, BPW)], idx_v)

    bufs = (buf0, buf1)
    sems = (sem0, sem1)

    def gather(g, j):
        off = pl.multiple_of(g * CH, CH)
        pltpu.async_copy(
            table_hbm.at[idx_v.at[pl.ds(off, CH)]], bufs[j], sems[j])

    def drain_and_store(g, j):
        pltpu.make_async_copy(table_hbm.at[pl.ds(0, CH)], bufs[j], sems[j]).wait()
        off = pl.multiple_of(g * CH, CH)
        pltpu.sync_copy(bufs[j], out_hbm.at[b, pl.ds(s0 + off, CH)])

    gather(0, 0)

    @pl.loop(0, NCHUNK, step=2)
    def _pair(g):
        gather(g + 1, 1)
        drain_and_store(g, 0)

        @pl.when(g + 2 < NCHUNK)
        def _():
            gather(g + 2, 0)

        drain_and_store(g + 1, 1)


def kernel(x, table):
    return _emb_lookup(x.astype(jnp.int32), table)
